# baseline (speedup 1.0000x reference)
import functools

import jax
import jax.numpy as jnp
from jax import lax
from jax.experimental import pallas as pl
from jax.experimental.pallas import tpu as pltpu
from jax.experimental.pallas import tpu_sc as plsc

N = 10000
E = 320000
D = 128
DE = 16
H = 128
L = 3
OUT = 128

BE = 2000


def _edge_body(ptf_ref, pef_ref, wn_ref, we_ref, b_ref, out_ref):
    x = ptf_ref[...]
    e = pef_ref[...]
    for l in range(L):
        t = (jnp.dot(x, wn_ref[l], preferred_element_type=jnp.float32)
             + jnp.dot(e, we_ref[l], preferred_element_type=jnp.float32)
             + b_ref[l])
        out_ref[l] = jnp.maximum(t, 0.0)


def _edge_transform(ptf, pef, wn, we, b, eoff, ecnt):
    ob = eoff // BE
    return pl.pallas_call(
        _edge_body,
        grid=(ecnt // BE,),
        in_specs=[
            pl.BlockSpec((BE, D), lambda e: (e + ob, 0)),
            pl.BlockSpec((BE, DE), lambda e: (e + ob, 0)),
            pl.BlockSpec((L, D, H), lambda e: (0, 0, 0)),
            pl.BlockSpec((L, DE, H), lambda e: (0, 0, 0)),
            pl.BlockSpec((L, 1, H), lambda e: (0, 0, 0)),
        ],
        out_specs=pl.BlockSpec((L, BE, H), lambda e: (0, e, 0)),
        out_shape=jax.ShapeDtypeStruct((L, ecnt, H), jnp.float32),
    )(ptf, pef, wn, we, b)


NS = 16
CHUNK = 80
RPT = 640
RCH = 80
NRC = RPT // RCH


NBUF = 3


def _sc_scatter_body(eoff, ecnt, init4d, t_hbm, idx_hbm, init_hbm, out_hbm,
                     idx_v0, idx_v1, idx_v2, rows_v0, rows_v1, rows_v2,
                     page_v, sem_i0, sem_i1, sem_i2, sem_r0, sem_r1, sem_r2,
                     sem_s0, sem_s1, sem_s2, acc_sh):
    EPC = ecnt // 2
    EPT = EPC // NS
    NCH = EPT // CHUNK
    cid = lax.axis_index("c")
    sid = lax.axis_index("s")
    tbase = cid * EPC + sid * EPT
    base = eoff + tbase
    idx_bufs = (idx_v0, idx_v1, idx_v2)
    row_bufs = (rows_v0, rows_v1, rows_v2)
    isems = (sem_i0, sem_i1, sem_i2)
    rsems = (sem_r0, sem_r1, sem_r2)
    ssems = (sem_s0, sem_s1, sem_s2)

    for l in range(L):
        def zbody(r, _):
            r0 = sid * RPT + r * RCH

            @pl.when(r0 < N)
            def _():
                if init4d:
                    pltpu.sync_copy(init_hbm.at[cid, l, pl.ds(r0, RCH), :],
                                    page_v)
                else:
                    pltpu.sync_copy(init_hbm.at[pl.ds(r0, RCH), :], page_v)
                pltpu.sync_copy(page_v, acc_sh.at[pl.ds(r0, RCH), :])

            return 0

        lax.fori_loop(0, NRC, zbody, 0)
        plsc.subcore_barrier()

        def _loads(j, b):
            return (
                pltpu.make_async_copy(idx_hbm.at[pl.ds(base + j * CHUNK,
                                                       CHUNK)],
                                      idx_bufs[b], isems[b]),
                pltpu.make_async_copy(t_hbm.at[l, pl.ds(tbase + j * CHUNK, CHUNK), :],
                                      row_bufs[b], rsems[b]),
            )

        def _scat(b):
            return pltpu.make_async_copy(row_bufs[b], acc_sh.at[idx_bufs[b]],
                                         ssems[b])

        for c in _loads(0, 0):
            c.start()
        for c in _loads(1, 1):
            c.start()

        def body(g, _):
            for b in range(NBUF):
                j = NBUF * g + b

                @pl.when(j < NCH)
                def _():
                    for c in _loads(j, b):
                        c.wait()

                    @pl.when(j >= 1)
                    def _():
                        _scat((b - 1) % NBUF).wait()

                    _scat(b).start(add=True)

                    @pl.when(j + 2 < NCH)
                    def _():
                        for c in _loads(j + 2, (b + 2) % NBUF):
                            c.start()

            return 0

        lax.fori_loop(0, (NCH + NBUF - 1) // NBUF, body, 0)
        _scat((NCH - 1) % NBUF).wait()
        plsc.subcore_barrier()

        def obody(r, _):
            r0 = sid * RPT + r * RCH

            @pl.when(r0 < N)
            def _():
                pltpu.sync_copy(acc_sh.at[pl.ds(r0, RCH), :], page_v)
                pltpu.sync_copy(page_v, out_hbm.at[cid, l, pl.ds(r0, RCH), :])

            return 0

        lax.fori_loop(0, NRC, obody, 0)
        plsc.subcore_barrier()


def _sc_scatter(t, idx, init, eoff, ecnt):
    mesh = plsc.VectorSubcoreMesh(core_axis_name="c", subcore_axis_name="s")
    init4d = init.ndim == 4
    f = functools.partial(
        pl.kernel,
        mesh=mesh,
        out_type=jax.ShapeDtypeStruct((2, L, N, H), jnp.float32),
        scratch_types=(
            [pltpu.VMEM((CHUNK,), jnp.int32)] * NBUF
            + [pltpu.VMEM((CHUNK, H), jnp.float32)] * NBUF
            + [pltpu.VMEM((RCH, H), jnp.float32)]
            + [pltpu.SemaphoreType.DMA] * (3 * NBUF)
            + [pltpu.VMEM_SHARED((N, H), jnp.float32)]
        ),
    )(functools.partial(_sc_scatter_body, eoff, ecnt, init4d))
    return f(t, idx, init)


def _node_body(x_ref, agg_ref, w1t_ref, g_ref, bta_ref, w2t_ref,
               wot_ref, bo_ref, out_ref):
    h = x_ref[...]
    acc = jnp.broadcast_to(bo_ref[...], (N, OUT))
    for i in range(L):
        agg_i = agg_ref[0, i] + agg_ref[1, i]
        u = h + agg_i
        h1 = jnp.dot(u, w1t_ref[i], preferred_element_type=jnp.float32)
        mean = jnp.mean(h1, axis=0, keepdims=True)
        var = jnp.mean((h1 - mean) ** 2, axis=0, keepdims=True)
        h1 = (h1 - mean) * lax.rsqrt(var + 1e-5) * g_ref[i] + bta_ref[i]
        h1 = jnp.maximum(h1, 0.0)
        h = jnp.dot(h1, w2t_ref[i], preferred_element_type=jnp.float32)
        acc = acc + jnp.dot(h, wot_ref[i], preferred_element_type=jnp.float32)
    out_ref[...] = acc


def _node_pipeline(x, agg, w1t, g, bta, w2t, wot, bo):
    return pl.pallas_call(
        _node_body,
        grid=(1,),
        in_specs=[
            pl.BlockSpec((N, D), lambda i: (0, 0)),
            pl.BlockSpec((2, L, N, H), lambda i: (0, 0, 0, 0)),
            pl.BlockSpec((L, H, H), lambda i: (0, 0, 0)),
            pl.BlockSpec((L, 1, H), lambda i: (0, 0, 0)),
            pl.BlockSpec((L, 1, H), lambda i: (0, 0, 0)),
            pl.BlockSpec((L, H, H), lambda i: (0, 0, 0)),
            pl.BlockSpec((L, H, OUT), lambda i: (0, 0, 0)),
            pl.BlockSpec((1, OUT), lambda i: (0, 0)),
        ],
        out_specs=pl.BlockSpec((N, OUT), lambda i: (0, 0)),
        out_shape=jax.ShapeDtypeStruct((N, OUT), jnp.float32),
    )(x, agg, w1t, g, bta, w2t, wot, bo)


def kernel(x_src_unique, unique_src_to_process_indices,
           path_source_indices_global_scatter, path_target_node_features,
           path_edge_features, We, be, W1, gamma, beta, W2, Wout, bout):
    wn = jnp.swapaxes(We[:, :, :D], 1, 2)
    we2 = jnp.swapaxes(We[:, :, D:], 1, 2)
    b = be.reshape(L, 1, H)
    zeros = jnp.zeros((N, H), dtype=jnp.float32)

    EA = 192000
    idx = path_source_indices_global_scatter
    t_a = _edge_transform(path_target_node_features, path_edge_features,
                          wn, we2, b, 0, EA)
    agg_a = _sc_scatter(t_a, idx, zeros, 0, EA)
    t_b = _edge_transform(path_target_node_features, path_edge_features,
                          wn, we2, b, EA, E - EA)
    agg = _sc_scatter(t_b, idx, agg_a, EA, E - EA)

    w1t = jnp.swapaxes(W1, 1, 2)
    w2t = jnp.swapaxes(W2, 1, 2)
    wot = Wout.T.reshape(L, H, OUT)
    final = _node_pipeline(x_src_unique, agg, w1t,
                           gamma.reshape(L, 1, H), beta.reshape(L, 1, H),
                           w2t, wot, bout.reshape(1, OUT))
    return (final, unique_src_to_process_indices)


# speedup vs baseline: 1.0636x; 1.0636x over previous
import functools

import jax
import jax.numpy as jnp
from jax import lax
from jax.experimental import pallas as pl
from jax.experimental.pallas import tpu as pltpu
from jax.experimental.pallas import tpu_sc as plsc

N = 10000
E = 320000
D = 128
DE = 16
H = 128
L = 3
OUT = 128

BE = 2000


def _edge_body(ptf_ref, pef_ref, wn_ref, we_ref, b_ref, out_ref):
    x = ptf_ref[...]
    e = pef_ref[...]
    for l in range(L):
        t = (jnp.dot(x, wn_ref[l], preferred_element_type=jnp.float32)
             + jnp.dot(e, we_ref[l], preferred_element_type=jnp.float32)
             + b_ref[l])
        out_ref[l] = jnp.maximum(t, 0.0)


def _edge_transform(ptf, pef, wn, we, b):
    return pl.pallas_call(
        _edge_body,
        grid=(E // BE,),
        in_specs=[
            pl.BlockSpec((BE, D), lambda e: (e, 0)),
            pl.BlockSpec((BE, DE), lambda e: (e, 0)),
            pl.BlockSpec((L, D, H), lambda e: (0, 0, 0)),
            pl.BlockSpec((L, DE, H), lambda e: (0, 0, 0)),
            pl.BlockSpec((L, 1, H), lambda e: (0, 0, 0)),
        ],
        out_specs=pl.BlockSpec((L, BE, H), lambda e: (0, e, 0)),
        out_shape=jax.ShapeDtypeStruct((L, E, H), jnp.float32),
    )(ptf, pef, wn, we, b)


NS = 16
EPC = E // 2
EPT = EPC // NS
CHUNK = 80
NCH = EPT // CHUNK
RPT = 640
RCH = 80
NRC = RPT // RCH


NBUF = 3


def _sc_scatter_body(t_hbm, idx_hbm, out_hbm,
                     idx_v0, idx_v1, idx_v2, rows_v0, rows_v1, rows_v2,
                     zpage_v, sem_i0, sem_i1, sem_i2, sem_r0, sem_r1, sem_r2,
                     sem_s0, sem_s1, sem_s2, sem_z, acc_sh):
    cid = lax.axis_index("c")
    sid = lax.axis_index("s")
    base = cid * EPC + sid * EPT
    idx_bufs = (idx_v0, idx_v1, idx_v2)
    row_bufs = (rows_v0, rows_v1, rows_v2)
    isems = (sem_i0, sem_i1, sem_i2)
    rsems = (sem_r0, sem_r1, sem_r2)
    ssems = (sem_s0, sem_s1, sem_s2)

    def zrow(r, _):
        def zcol(c, _):
            zpage_v[r, pl.ds(c * 16, 16)] = jnp.zeros((16,), jnp.float32)
            return 0

        lax.fori_loop(0, H // 16, zcol, 0)
        return 0

    lax.fori_loop(0, RCH, zrow, 0)

    def _initcp(r):
        r0 = sid * RPT + r * RCH
        return pltpu.make_async_copy(zpage_v, acc_sh.at[pl.ds(r0, RCH), :],
                                     sem_z)

    def _draincp(l, r):
        r0 = sid * RPT + r * RCH
        return pltpu.make_async_copy(acc_sh.at[pl.ds(r0, RCH), :],
                                     out_hbm.at[cid, l, pl.ds(r0, RCH), :],
                                     sem_z)

    for l in range(L):
        for r in range(NRC):
            @pl.when(sid * RPT + r * RCH < N)
            def _(r=r):
                _initcp(r).start()
        for r in range(NRC):
            @pl.when(sid * RPT + r * RCH < N)
            def _(r=r):
                _initcp(r).wait()
        plsc.subcore_barrier()

        def _loads(j, b):
            off = base + j * CHUNK
            return (
                pltpu.make_async_copy(idx_hbm.at[pl.ds(off, CHUNK)],
                                      idx_bufs[b], isems[b]),
                pltpu.make_async_copy(t_hbm.at[l, pl.ds(off, CHUNK), :],
                                      row_bufs[b], rsems[b]),
            )

        def _scat(b):
            return pltpu.make_async_copy(row_bufs[b], acc_sh.at[idx_bufs[b]],
                                         ssems[b])

        for c in _loads(0, 0):
            c.start()
        for c in _loads(1, 1):
            c.start()

        def body(g, _):
            for b in range(NBUF):
                j = NBUF * g + b

                @pl.when(j < NCH)
                def _():
                    for c in _loads(j, b):
                        c.wait()

                    @pl.when(j >= 1)
                    def _():
                        _scat((b - 1) % NBUF).wait()

                    _scat(b).start(add=True)

                    @pl.when(j + 2 < NCH)
                    def _():
                        for c in _loads(j + 2, (b + 2) % NBUF):
                            c.start()

            return 0

        lax.fori_loop(0, (NCH + NBUF - 1) // NBUF, body, 0)
        _scat((NCH - 1) % NBUF).wait()
        plsc.subcore_barrier()

        for r in range(NRC):
            @pl.when(sid * RPT + r * RCH < N)
            def _(r=r, l=l):
                _draincp(l, r).start()
        for r in range(NRC):
            @pl.when(sid * RPT + r * RCH < N)
            def _(r=r, l=l):
                _draincp(l, r).wait()
        plsc.subcore_barrier()


def _sc_scatter(t, idx):
    mesh = plsc.VectorSubcoreMesh(core_axis_name="c", subcore_axis_name="s")
    f = functools.partial(
        pl.kernel,
        mesh=mesh,
        out_type=jax.ShapeDtypeStruct((2, L, N, H), jnp.float32),
        scratch_types=(
            [pltpu.VMEM((CHUNK,), jnp.int32)] * NBUF
            + [pltpu.VMEM((CHUNK, H), jnp.float32)] * NBUF
            + [pltpu.VMEM((RCH, H), jnp.float32)]
            + [pltpu.SemaphoreType.DMA] * (3 * NBUF + 1)
            + [pltpu.VMEM_SHARED((N, H), jnp.float32)]
        ),
    )(_sc_scatter_body)
    return f(t, idx)


def _node_body(x_ref, agg_ref, w1t_ref, g_ref, bta_ref, w2t_ref, wot_ref,
               bo_ref, out_ref):
    h = x_ref[...]
    acc = jnp.broadcast_to(bo_ref[...], (N, OUT))
    for i in range(L):
        agg_i = agg_ref[0, i] + agg_ref[1, i]
        u = h + agg_i
        h1 = jnp.dot(u, w1t_ref[i], preferred_element_type=jnp.float32)
        mean = jnp.mean(h1, axis=0, keepdims=True)
        var = jnp.mean((h1 - mean) ** 2, axis=0, keepdims=True)
        h1 = (h1 - mean) * lax.rsqrt(var + 1e-5) * g_ref[i] + bta_ref[i]
        h1 = jnp.maximum(h1, 0.0)
        h = jnp.dot(h1, w2t_ref[i], preferred_element_type=jnp.float32)
        acc = acc + jnp.dot(h, wot_ref[i], preferred_element_type=jnp.float32)
    out_ref[...] = acc


def _node_pipeline(x, agg, w1t, g, bta, w2t, wot, bo):
    return pl.pallas_call(
        _node_body,
        grid=(1,),
        in_specs=[
            pl.BlockSpec((N, D), lambda i: (0, 0)),
            pl.BlockSpec((2, L, N, H), lambda i: (0, 0, 0, 0)),
            pl.BlockSpec((L, H, H), lambda i: (0, 0, 0)),
            pl.BlockSpec((L, 1, H), lambda i: (0, 0, 0)),
            pl.BlockSpec((L, 1, H), lambda i: (0, 0, 0)),
            pl.BlockSpec((L, H, H), lambda i: (0, 0, 0)),
            pl.BlockSpec((L, H, OUT), lambda i: (0, 0, 0)),
            pl.BlockSpec((1, OUT), lambda i: (0, 0)),
        ],
        out_specs=pl.BlockSpec((N, OUT), lambda i: (0, 0)),
        out_shape=jax.ShapeDtypeStruct((N, OUT), jnp.float32),
    )(x, agg, w1t, g, bta, w2t, wot, bo)


def kernel(x_src_unique, unique_src_to_process_indices,
           path_source_indices_global_scatter, path_target_node_features,
           path_edge_features, We, be, W1, gamma, beta, W2, Wout, bout):
    wn = jnp.swapaxes(We[:, :, :D], 1, 2)
    we2 = jnp.swapaxes(We[:, :, D:], 1, 2)
    b = be.reshape(L, 1, H)

    t = _edge_transform(path_target_node_features, path_edge_features,
                        wn, we2, b)

    agg = _sc_scatter(t, path_source_indices_global_scatter)

    w1t = jnp.swapaxes(W1, 1, 2)
    w2t = jnp.swapaxes(W2, 1, 2)
    wot = Wout.T.reshape(L, H, OUT)
    final = _node_pipeline(x_src_unique, agg, w1t,
                           gamma.reshape(L, 1, H), beta.reshape(L, 1, H),
                           w2t, wot, bout.reshape(1, OUT))
    return (final, unique_src_to_process_indices)


# speedup vs baseline: 1.1254x; 1.0582x over previous
import functools

import jax
import jax.numpy as jnp
from jax import lax
from jax.experimental import pallas as pl
from jax.experimental.pallas import tpu as pltpu
from jax.experimental.pallas import tpu_sc as plsc

N = 10000
E = 320000
D = 128
DE = 16
H = 128
L = 3
OUT = 128

BE = 4000


def _edge_body(ptf_ref, pef_ref, wn_ref, we_ref, b_ref, out_ref):
    x = ptf_ref[...]
    e = pef_ref[...]
    for l in range(L):
        t = (jnp.dot(x, wn_ref[l], preferred_element_type=jnp.float32)
             + jnp.dot(e, we_ref[l], preferred_element_type=jnp.float32)
             + b_ref[l])
        out_ref[l] = jnp.maximum(t, 0.0)


def _edge_transform(ptf, pef, wn, we, b):
    return pl.pallas_call(
        _edge_body,
        grid=(E // BE,),
        in_specs=[
            pl.BlockSpec((BE, D), lambda e: (e, 0)),
            pl.BlockSpec((BE, DE), lambda e: (e, 0)),
            pl.BlockSpec((L, D, H), lambda e: (0, 0, 0)),
            pl.BlockSpec((L, DE, H), lambda e: (0, 0, 0)),
            pl.BlockSpec((L, 1, H), lambda e: (0, 0, 0)),
        ],
        out_specs=pl.BlockSpec((L, BE, H), lambda e: (0, e, 0)),
        out_shape=jax.ShapeDtypeStruct((L, E, H), jnp.float32),
    )(ptf, pef, wn, we, b)


NS = 16
EPC = E // 2
EPT = EPC // NS
CHUNK = 80
NCH = EPT // CHUNK
RPT = 640
RCH = 80
NRC = RPT // RCH


NBUF = 3


def _sc_scatter_body(t_hbm, idx_hbm, out_hbm,
                     idx_v0, idx_v1, idx_v2, rows_v0, rows_v1, rows_v2,
                     zpage_v, sem_i0, sem_i1, sem_i2, sem_r0, sem_r1, sem_r2,
                     sem_s0, sem_s1, sem_s2, sem_z, acc_sh):
    cid = lax.axis_index("c")
    sid = lax.axis_index("s")
    base = cid * EPC + sid * EPT
    idx_bufs = (idx_v0, idx_v1, idx_v2)
    row_bufs = (rows_v0, rows_v1, rows_v2)
    isems = (sem_i0, sem_i1, sem_i2)
    rsems = (sem_r0, sem_r1, sem_r2)
    ssems = (sem_s0, sem_s1, sem_s2)

    def zrow(r, _):
        def zcol(c, _):
            zpage_v[r, pl.ds(c * 16, 16)] = jnp.zeros((16,), jnp.float32)
            return 0

        lax.fori_loop(0, H // 16, zcol, 0)
        return 0

    lax.fori_loop(0, RCH, zrow, 0)

    def _initcp(r):
        r0 = sid * RPT + r * RCH
        return pltpu.make_async_copy(zpage_v, acc_sh.at[pl.ds(r0, RCH), :],
                                     sem_z)

    def _draincp(l, r):
        r0 = sid * RPT + r * RCH
        return pltpu.make_async_copy(acc_sh.at[pl.ds(r0, RCH), :],
                                     out_hbm.at[cid, l, pl.ds(r0, RCH), :],
                                     sem_z)

    for l in range(L):
        for r in range(NRC):
            @pl.when(sid * RPT + r * RCH < N)
            def _(r=r):
                _initcp(r).start()
        for r in range(NRC):
            @pl.when(sid * RPT + r * RCH < N)
            def _(r=r):
                _initcp(r).wait()
        plsc.subcore_barrier()

        def _loads(j, b):
            off = base + j * CHUNK
            return (
                pltpu.make_async_copy(idx_hbm.at[pl.ds(off, CHUNK)],
                                      idx_bufs[b], isems[b]),
                pltpu.make_async_copy(t_hbm.at[l, pl.ds(off, CHUNK), :],
                                      row_bufs[b], rsems[b]),
            )

        def _scat(b):
            return pltpu.make_async_copy(row_bufs[b], acc_sh.at[idx_bufs[b]],
                                         ssems[b])

        for c in _loads(0, 0):
            c.start()
        for c in _loads(1, 1):
            c.start()

        def body(g, _):
            for b in range(NBUF):
                j = NBUF * g + b

                @pl.when(j < NCH)
                def _():
                    for c in _loads(j, b):
                        c.wait()

                    @pl.when(j >= 1)
                    def _():
                        _scat((b - 1) % NBUF).wait()

                    _scat(b).start(add=True)

                    @pl.when(j + 2 < NCH)
                    def _():
                        for c in _loads(j + 2, (b + 2) % NBUF):
                            c.start()

            return 0

        lax.fori_loop(0, (NCH + NBUF - 1) // NBUF, body, 0)
        _scat((NCH - 1) % NBUF).wait()
        plsc.subcore_barrier()

        for r in range(NRC):
            @pl.when(sid * RPT + r * RCH < N)
            def _(r=r, l=l):
                _draincp(l, r).start()
        for r in range(NRC):
            @pl.when(sid * RPT + r * RCH < N)
            def _(r=r, l=l):
                _draincp(l, r).wait()
        plsc.subcore_barrier()


def _sc_scatter(t, idx):
    mesh = plsc.VectorSubcoreMesh(core_axis_name="c", subcore_axis_name="s")
    f = functools.partial(
        pl.kernel,
        mesh=mesh,
        out_type=jax.ShapeDtypeStruct((2, L, N, H), jnp.float32),
        scratch_types=(
            [pltpu.VMEM((CHUNK,), jnp.int32)] * NBUF
            + [pltpu.VMEM((CHUNK, H), jnp.float32)] * NBUF
            + [pltpu.VMEM((RCH, H), jnp.float32)]
            + [pltpu.SemaphoreType.DMA] * (3 * NBUF + 1)
            + [pltpu.VMEM_SHARED((N, H), jnp.float32)]
        ),
    )(_sc_scatter_body)
    return f(t, idx)


def _node_body(x_ref, agg_ref, w1t_ref, g_ref, bta_ref, w2t_ref, wot_ref,
               bo_ref, out_ref):
    h = x_ref[...]
    acc = jnp.broadcast_to(bo_ref[...], (N, OUT))
    for i in range(L):
        agg_i = agg_ref[0, i] + agg_ref[1, i]
        u = h + agg_i
        h1 = jnp.dot(u, w1t_ref[i], preferred_element_type=jnp.float32)
        mean = jnp.mean(h1, axis=0, keepdims=True)
        var = jnp.mean(h1 * h1, axis=0, keepdims=True) - mean * mean
        h1 = (h1 - mean) * lax.rsqrt(var + 1e-5) * g_ref[i] + bta_ref[i]
        h1 = jnp.maximum(h1, 0.0)
        h = jnp.dot(h1, w2t_ref[i], preferred_element_type=jnp.float32)
        acc = acc + jnp.dot(h, wot_ref[i], preferred_element_type=jnp.float32)
    out_ref[...] = acc


def _node_pipeline(x, agg, w1t, g, bta, w2t, wot, bo):
    return pl.pallas_call(
        _node_body,
        grid=(1,),
        in_specs=[
            pl.BlockSpec((N, D), lambda i: (0, 0)),
            pl.BlockSpec((2, L, N, H), lambda i: (0, 0, 0, 0)),
            pl.BlockSpec((L, H, H), lambda i: (0, 0, 0)),
            pl.BlockSpec((L, 1, H), lambda i: (0, 0, 0)),
            pl.BlockSpec((L, 1, H), lambda i: (0, 0, 0)),
            pl.BlockSpec((L, H, H), lambda i: (0, 0, 0)),
            pl.BlockSpec((L, H, OUT), lambda i: (0, 0, 0)),
            pl.BlockSpec((1, OUT), lambda i: (0, 0)),
        ],
        out_specs=pl.BlockSpec((N, OUT), lambda i: (0, 0)),
        out_shape=jax.ShapeDtypeStruct((N, OUT), jnp.float32),
    )(x, agg, w1t, g, bta, w2t, wot, bo)


def kernel(x_src_unique, unique_src_to_process_indices,
           path_source_indices_global_scatter, path_target_node_features,
           path_edge_features, We, be, W1, gamma, beta, W2, Wout, bout):
    wn = jnp.swapaxes(We[:, :, :D], 1, 2)
    we2 = jnp.swapaxes(We[:, :, D:], 1, 2)
    b = be.reshape(L, 1, H)

    t = _edge_transform(path_target_node_features, path_edge_features,
                        wn, we2, b)

    agg = _sc_scatter(t, path_source_indices_global_scatter)

    w1t = jnp.swapaxes(W1, 1, 2)
    w2t = jnp.swapaxes(W2, 1, 2)
    wot = Wout.T.reshape(L, H, OUT)
    final = _node_pipeline(x_src_unique, agg, w1t,
                           gamma.reshape(L, 1, H), beta.reshape(L, 1, H),
                           w2t, wot, bout.reshape(1, OUT))
    return (final, unique_src_to_process_indices)


# speedup vs baseline: 1.1380x; 1.0112x over previous
import functools

import jax
import jax.numpy as jnp
from jax import lax
from jax.experimental import pallas as pl
from jax.experimental.pallas import tpu as pltpu
from jax.experimental.pallas import tpu_sc as plsc

N = 10000
E = 320000
D = 128
DE = 16
H = 128
L = 3
OUT = 128

BE = 8000


def _edge_body(ptf_ref, pef_ref, wn_ref, we_ref, b_ref, out_ref):
    x = ptf_ref[...]
    e = pef_ref[...]
    for l in range(L):
        t = (jnp.dot(x, wn_ref[l], preferred_element_type=jnp.float32)
             + jnp.dot(e, we_ref[l], preferred_element_type=jnp.float32)
             + b_ref[l])
        out_ref[l] = jnp.maximum(t, 0.0)


def _edge_transform(ptf, pef, wn, we, b):
    return pl.pallas_call(
        _edge_body,
        grid=(E // BE,),
        in_specs=[
            pl.BlockSpec((BE, D), lambda e: (e, 0)),
            pl.BlockSpec((BE, DE), lambda e: (e, 0)),
            pl.BlockSpec((L, D, H), lambda e: (0, 0, 0)),
            pl.BlockSpec((L, DE, H), lambda e: (0, 0, 0)),
            pl.BlockSpec((L, 1, H), lambda e: (0, 0, 0)),
        ],
        out_specs=pl.BlockSpec((L, BE, H), lambda e: (0, e, 0)),
        out_shape=jax.ShapeDtypeStruct((L, E, H), jnp.float32),
    )(ptf, pef, wn, we, b)


NS = 16
EPC = E // 2
EPT = EPC // NS
CHUNK = 80
NCH = EPT // CHUNK
RPT = 640
RCH = 80
NRC = RPT // RCH


NBUF = 3


def _sc_scatter_body(t_hbm, idx_hbm, out_hbm,
                     idx_v0, idx_v1, idx_v2, rows_v0, rows_v1, rows_v2,
                     zpage_v, sem_i0, sem_i1, sem_i2, sem_r0, sem_r1, sem_r2,
                     sem_s0, sem_s1, sem_s2, sem_z, acc_sh):
    cid = lax.axis_index("c")
    sid = lax.axis_index("s")
    base = cid * EPC + sid * EPT
    idx_bufs = (idx_v0, idx_v1, idx_v2)
    row_bufs = (rows_v0, rows_v1, rows_v2)
    isems = (sem_i0, sem_i1, sem_i2)
    rsems = (sem_r0, sem_r1, sem_r2)
    ssems = (sem_s0, sem_s1, sem_s2)

    def zrow(r, _):
        def zcol(c, _):
            zpage_v[r, pl.ds(c * 16, 16)] = jnp.zeros((16,), jnp.float32)
            return 0

        lax.fori_loop(0, H // 16, zcol, 0)
        return 0

    lax.fori_loop(0, RCH, zrow, 0)

    def _initcp(r):
        r0 = sid * RPT + r * RCH
        return pltpu.make_async_copy(zpage_v, acc_sh.at[pl.ds(r0, RCH), :],
                                     sem_z)

    def _draincp(l, r):
        r0 = sid * RPT + r * RCH
        return pltpu.make_async_copy(acc_sh.at[pl.ds(r0, RCH), :],
                                     out_hbm.at[cid, l, pl.ds(r0, RCH), :],
                                     sem_z)

    for l in range(L):
        for r in range(NRC):
            @pl.when(sid * RPT + r * RCH < N)
            def _(r=r):
                _initcp(r).start()
        for r in range(NRC):
            @pl.when(sid * RPT + r * RCH < N)
            def _(r=r):
                _initcp(r).wait()
        plsc.subcore_barrier()

        def _loads(j, b):
            off = base + j * CHUNK
            return (
                pltpu.make_async_copy(idx_hbm.at[pl.ds(off, CHUNK)],
                                      idx_bufs[b], isems[b]),
                pltpu.make_async_copy(t_hbm.at[l, pl.ds(off, CHUNK), :],
                                      row_bufs[b], rsems[b]),
            )

        def _scat(b):
            return pltpu.make_async_copy(row_bufs[b], acc_sh.at[idx_bufs[b]],
                                         ssems[b])

        for c in _loads(0, 0):
            c.start()
        for c in _loads(1, 1):
            c.start()

        def body(g, _):
            for b in range(NBUF):
                j = NBUF * g + b

                @pl.when(j < NCH)
                def _():
                    for c in _loads(j, b):
                        c.wait()

                    @pl.when(j >= 1)
                    def _():
                        _scat((b - 1) % NBUF).wait()

                    _scat(b).start(add=True)

                    @pl.when(j + 2 < NCH)
                    def _():
                        for c in _loads(j + 2, (b + 2) % NBUF):
                            c.start()

            return 0

        lax.fori_loop(0, (NCH + NBUF - 1) // NBUF, body, 0)
        _scat((NCH - 1) % NBUF).wait()
        plsc.subcore_barrier()

        for r in range(NRC):
            @pl.when(sid * RPT + r * RCH < N)
            def _(r=r, l=l):
                _draincp(l, r).start()
        for r in range(NRC):
            @pl.when(sid * RPT + r * RCH < N)
            def _(r=r, l=l):
                _draincp(l, r).wait()
        plsc.subcore_barrier()


def _sc_scatter(t, idx):
    mesh = plsc.VectorSubcoreMesh(core_axis_name="c", subcore_axis_name="s")
    f = functools.partial(
        pl.kernel,
        mesh=mesh,
        out_type=jax.ShapeDtypeStruct((2, L, N, H), jnp.float32),
        scratch_types=(
            [pltpu.VMEM((CHUNK,), jnp.int32)] * NBUF
            + [pltpu.VMEM((CHUNK, H), jnp.float32)] * NBUF
            + [pltpu.VMEM((RCH, H), jnp.float32)]
            + [pltpu.SemaphoreType.DMA] * (3 * NBUF + 1)
            + [pltpu.VMEM_SHARED((N, H), jnp.float32)]
        ),
    )(_sc_scatter_body)
    return f(t, idx)


def _node_body(x_ref, agg_ref, w1t_ref, g_ref, bta_ref, w2t_ref, wot_ref,
               bo_ref, out_ref):
    h = x_ref[...]
    acc = jnp.broadcast_to(bo_ref[...], (N, OUT))
    for i in range(L):
        agg_i = agg_ref[0, i] + agg_ref[1, i]
        u = h + agg_i
        h1 = jnp.dot(u, w1t_ref[i], preferred_element_type=jnp.float32)
        mean = jnp.mean(h1, axis=0, keepdims=True)
        var = jnp.mean(h1 * h1, axis=0, keepdims=True) - mean * mean
        h1 = (h1 - mean) * lax.rsqrt(var + 1e-5) * g_ref[i] + bta_ref[i]
        h1 = jnp.maximum(h1, 0.0)
        h = jnp.dot(h1, w2t_ref[i], preferred_element_type=jnp.float32)
        acc = acc + jnp.dot(h, wot_ref[i], preferred_element_type=jnp.float32)
    out_ref[...] = acc


def _node_pipeline(x, agg, w1t, g, bta, w2t, wot, bo):
    return pl.pallas_call(
        _node_body,
        grid=(1,),
        in_specs=[
            pl.BlockSpec((N, D), lambda i: (0, 0)),
            pl.BlockSpec((2, L, N, H), lambda i: (0, 0, 0, 0)),
            pl.BlockSpec((L, H, H), lambda i: (0, 0, 0)),
            pl.BlockSpec((L, 1, H), lambda i: (0, 0, 0)),
            pl.BlockSpec((L, 1, H), lambda i: (0, 0, 0)),
            pl.BlockSpec((L, H, H), lambda i: (0, 0, 0)),
            pl.BlockSpec((L, H, OUT), lambda i: (0, 0, 0)),
            pl.BlockSpec((1, OUT), lambda i: (0, 0)),
        ],
        out_specs=pl.BlockSpec((N, OUT), lambda i: (0, 0)),
        out_shape=jax.ShapeDtypeStruct((N, OUT), jnp.float32),
    )(x, agg, w1t, g, bta, w2t, wot, bo)


def kernel(x_src_unique, unique_src_to_process_indices,
           path_source_indices_global_scatter, path_target_node_features,
           path_edge_features, We, be, W1, gamma, beta, W2, Wout, bout):
    wn = jnp.swapaxes(We[:, :, :D], 1, 2)
    we2 = jnp.swapaxes(We[:, :, D:], 1, 2)
    b = be.reshape(L, 1, H)

    t = _edge_transform(path_target_node_features, path_edge_features,
                        wn, we2, b)

    agg = _sc_scatter(t, path_source_indices_global_scatter)

    w1t = jnp.swapaxes(W1, 1, 2)
    w2t = jnp.swapaxes(W2, 1, 2)
    wot = Wout.T.reshape(L, H, OUT)
    final = _node_pipeline(x_src_unique, agg, w1t,
                           gamma.reshape(L, 1, H), beta.reshape(L, 1, H),
                           w2t, wot, bout.reshape(1, OUT))
    return (final, unique_src_to_process_indices)


# speedup vs baseline: 1.1409x; 1.0025x over previous
import functools

import jax
import jax.numpy as jnp
from jax import lax
from jax.experimental import pallas as pl
from jax.experimental.pallas import tpu as pltpu
from jax.experimental.pallas import tpu_sc as plsc

N = 10000
E = 320000
D = 128
DE = 16
H = 128
L = 3
OUT = 128

BE = 10000


def _edge_body(ptf_ref, pef_ref, wn_ref, we_ref, b_ref, out_ref):
    x = ptf_ref[...]
    e = pef_ref[...]
    for l in range(L):
        t = (jnp.dot(x, wn_ref[l], preferred_element_type=jnp.float32)
             + jnp.dot(e, we_ref[l], preferred_element_type=jnp.float32)
             + b_ref[l])
        out_ref[l] = jnp.maximum(t, 0.0)


def _edge_transform(ptf, pef, wn, we, b):
    return pl.pallas_call(
        _edge_body,
        grid=(E // BE,),
        in_specs=[
            pl.BlockSpec((BE, D), lambda e: (e, 0)),
            pl.BlockSpec((BE, DE), lambda e: (e, 0)),
            pl.BlockSpec((L, D, H), lambda e: (0, 0, 0)),
            pl.BlockSpec((L, DE, H), lambda e: (0, 0, 0)),
            pl.BlockSpec((L, 1, H), lambda e: (0, 0, 0)),
        ],
        out_specs=pl.BlockSpec((L, BE, H), lambda e: (0, e, 0)),
        out_shape=jax.ShapeDtypeStruct((L, E, H), jnp.float32),
    )(ptf, pef, wn, we, b)


NS = 16
EPC = E // 2
EPT = EPC // NS
CHUNK = 80
NCH = EPT // CHUNK
RPT = 640
RCH = 80
NRC = RPT // RCH


NBUF = 3


def _sc_scatter_body(t_hbm, idx_hbm, out_hbm,
                     idx_v0, idx_v1, idx_v2, rows_v0, rows_v1, rows_v2,
                     zpage_v, sem_i0, sem_i1, sem_i2, sem_r0, sem_r1, sem_r2,
                     sem_s0, sem_s1, sem_s2, sem_z, acc_sh):
    cid = lax.axis_index("c")
    sid = lax.axis_index("s")
    base = cid * EPC + sid * EPT
    idx_bufs = (idx_v0, idx_v1, idx_v2)
    row_bufs = (rows_v0, rows_v1, rows_v2)
    isems = (sem_i0, sem_i1, sem_i2)
    rsems = (sem_r0, sem_r1, sem_r2)
    ssems = (sem_s0, sem_s1, sem_s2)

    def zrow(r, _):
        def zcol(c, _):
            zpage_v[r, pl.ds(c * 16, 16)] = jnp.zeros((16,), jnp.float32)
            return 0

        lax.fori_loop(0, H // 16, zcol, 0)
        return 0

    lax.fori_loop(0, RCH, zrow, 0)

    def _initcp(r):
        r0 = sid * RPT + r * RCH
        return pltpu.make_async_copy(zpage_v, acc_sh.at[pl.ds(r0, RCH), :],
                                     sem_z)

    def _draincp(l, r):
        r0 = sid * RPT + r * RCH
        return pltpu.make_async_copy(acc_sh.at[pl.ds(r0, RCH), :],
                                     out_hbm.at[cid, l, pl.ds(r0, RCH), :],
                                     sem_z)

    for l in range(L):
        for r in range(NRC):
            @pl.when(sid * RPT + r * RCH < N)
            def _(r=r):
                _initcp(r).start()
        for r in range(NRC):
            @pl.when(sid * RPT + r * RCH < N)
            def _(r=r):
                _initcp(r).wait()
        plsc.subcore_barrier()

        def _loads(j, b):
            off = base + j * CHUNK
            return (
                pltpu.make_async_copy(idx_hbm.at[pl.ds(off, CHUNK)],
                                      idx_bufs[b], isems[b]),
                pltpu.make_async_copy(t_hbm.at[l, pl.ds(off, CHUNK), :],
                                      row_bufs[b], rsems[b]),
            )

        def _scat(b):
            return pltpu.make_async_copy(row_bufs[b], acc_sh.at[idx_bufs[b]],
                                         ssems[b])

        for c in _loads(0, 0):
            c.start()
        for c in _loads(1, 1):
            c.start()

        def body(g, _):
            for b in range(NBUF):
                j = NBUF * g + b

                @pl.when(j < NCH)
                def _():
                    for c in _loads(j, b):
                        c.wait()

                    @pl.when(j >= 1)
                    def _():
                        _scat((b - 1) % NBUF).wait()

                    _scat(b).start(add=True)

                    @pl.when(j + 2 < NCH)
                    def _():
                        for c in _loads(j + 2, (b + 2) % NBUF):
                            c.start()

            return 0

        lax.fori_loop(0, (NCH + NBUF - 1) // NBUF, body, 0)
        _scat((NCH - 1) % NBUF).wait()
        plsc.subcore_barrier()

        for r in range(NRC):
            @pl.when(sid * RPT + r * RCH < N)
            def _(r=r, l=l):
                _draincp(l, r).start()
        for r in range(NRC):
            @pl.when(sid * RPT + r * RCH < N)
            def _(r=r, l=l):
                _draincp(l, r).wait()
        plsc.subcore_barrier()


def _sc_scatter(t, idx):
    mesh = plsc.VectorSubcoreMesh(core_axis_name="c", subcore_axis_name="s")
    f = functools.partial(
        pl.kernel,
        mesh=mesh,
        out_type=jax.ShapeDtypeStruct((2, L, N, H), jnp.float32),
        scratch_types=(
            [pltpu.VMEM((CHUNK,), jnp.int32)] * NBUF
            + [pltpu.VMEM((CHUNK, H), jnp.float32)] * NBUF
            + [pltpu.VMEM((RCH, H), jnp.float32)]
            + [pltpu.SemaphoreType.DMA] * (3 * NBUF + 1)
            + [pltpu.VMEM_SHARED((N, H), jnp.float32)]
        ),
    )(_sc_scatter_body)
    return f(t, idx)


def _node_body(x_ref, agg_ref, w1t_ref, g_ref, bta_ref, w2t_ref, wot_ref,
               bo_ref, out_ref):
    h = x_ref[...]
    acc = jnp.broadcast_to(bo_ref[...], (N, OUT))
    for i in range(L):
        agg_i = agg_ref[0, i] + agg_ref[1, i]
        u = h + agg_i
        h1 = jnp.dot(u, w1t_ref[i], preferred_element_type=jnp.float32)
        mean = jnp.mean(h1, axis=0, keepdims=True)
        var = jnp.mean(h1 * h1, axis=0, keepdims=True) - mean * mean
        h1 = (h1 - mean) * lax.rsqrt(var + 1e-5) * g_ref[i] + bta_ref[i]
        h1 = jnp.maximum(h1, 0.0)
        h = jnp.dot(h1, w2t_ref[i], preferred_element_type=jnp.float32)
        acc = acc + jnp.dot(h, wot_ref[i], preferred_element_type=jnp.float32)
    out_ref[...] = acc


def _node_pipeline(x, agg, w1t, g, bta, w2t, wot, bo):
    return pl.pallas_call(
        _node_body,
        grid=(1,),
        in_specs=[
            pl.BlockSpec((N, D), lambda i: (0, 0)),
            pl.BlockSpec((2, L, N, H), lambda i: (0, 0, 0, 0)),
            pl.BlockSpec((L, H, H), lambda i: (0, 0, 0)),
            pl.BlockSpec((L, 1, H), lambda i: (0, 0, 0)),
            pl.BlockSpec((L, 1, H), lambda i: (0, 0, 0)),
            pl.BlockSpec((L, H, H), lambda i: (0, 0, 0)),
            pl.BlockSpec((L, H, OUT), lambda i: (0, 0, 0)),
            pl.BlockSpec((1, OUT), lambda i: (0, 0)),
        ],
        out_specs=pl.BlockSpec((N, OUT), lambda i: (0, 0)),
        out_shape=jax.ShapeDtypeStruct((N, OUT), jnp.float32),
    )(x, agg, w1t, g, bta, w2t, wot, bo)


def kernel(x_src_unique, unique_src_to_process_indices,
           path_source_indices_global_scatter, path_target_node_features,
           path_edge_features, We, be, W1, gamma, beta, W2, Wout, bout):
    wn = jnp.swapaxes(We[:, :, :D], 1, 2)
    we2 = jnp.swapaxes(We[:, :, D:], 1, 2)
    b = be.reshape(L, 1, H)

    t = _edge_transform(path_target_node_features, path_edge_features,
                        wn, we2, b)

    agg = _sc_scatter(t, path_source_indices_global_scatter)

    w1t = jnp.swapaxes(W1, 1, 2)
    w2t = jnp.swapaxes(W2, 1, 2)
    wot = Wout.T.reshape(L, H, OUT)
    final = _node_pipeline(x_src_unique, agg, w1t,
                           gamma.reshape(L, 1, H), beta.reshape(L, 1, H),
                           w2t, wot, bout.reshape(1, OUT))
    return (final, unique_src_to_process_indices)
